# baseline (speedup 1.0000x reference)
import jax
import jax.numpy as jnp
from jax.experimental import pallas as pl

_TEMPERATURE = 1.0
_EPS = 1e-10
_BR = 8


def _gs_body(x_ref, u_ref, o_ref):
    u = u_ref[...]
    g = -jnp.log(-jnp.log(u + _EPS) + _EPS)
    y = x_ref[...] + g
    m = jnp.max(y, axis=-1, keepdims=True)
    e = jnp.exp(y - m)
    s = jnp.sum(e, axis=-1, keepdims=True)
    o_ref[...] = e / s


def kernel(logits):
    rows, cols = logits.shape
    u = jax.random.uniform(jax.random.key(42), logits.shape, dtype=logits.dtype)
    spec = pl.BlockSpec((_BR, cols), lambda i: (i, 0))
    return pl.pallas_call(
        _gs_body,
        grid=(rows // _BR,),
        in_specs=[spec, spec],
        out_specs=spec,
        out_shape=jax.ShapeDtypeStruct((rows, cols), logits.dtype),
    )(logits, u)
